# Initial kernel scaffold; baseline (speedup 1.0000x reference)
import functools

import jax
import jax.numpy as jnp
from jax import lax
from jax.experimental import pallas as pl
from jax.experimental.pallas import tpu as pltpu
from jax.experimental.pallas import tpu_sc as plsc

_H = 64


@functools.lru_cache(maxsize=None)
def _build_sc_kernel(N, RB):
    NW = 32
    per_w = N // NW
    n_chunks = per_w // RB
    assert per_w % RB == 0

    mesh = plsc.VectorSubcoreMesh(core_axis_name="c", subcore_axis_name="s")

    @functools.partial(
        pl.kernel,
        mesh=mesh,
        out_type=jax.ShapeDtypeStruct((N, _H), jnp.float32),
        scratch_types=[
            pltpu.VMEM((RB,), jnp.int32),
            pltpu.VMEM((RB,), jnp.int32),
            pltpu.VMEM((RB,), jnp.int32),
            pltpu.VMEM((RB, _H), jnp.float32),
            pltpu.SemaphoreType.DMA,
        ],
    )
    def sc_kernel(qid, cid, pid, qtab, ctab, ptab, out, qi_v, ci_v, pi_v,
                  buf, sem):
        wid = lax.axis_index("s") * 2 + lax.axis_index("c")
        base = wid * per_w

        def chunk(g, carry):
            off = base + g * RB
            pltpu.sync_copy(qid.at[pl.ds(off, RB)], qi_v)
            pltpu.sync_copy(cid.at[pl.ds(off, RB)], ci_v)
            pltpu.sync_copy(pid.at[pl.ds(off, RB)], pi_v)
            pltpu.async_copy(ctab.at[ci_v], buf, sem).wait()
            pltpu.async_copy(qtab.at[qi_v], buf, sem, add=True).wait()
            pltpu.async_copy(ptab.at[pi_v], buf, sem, add=True).wait()
            pltpu.sync_copy(buf, out.at[pl.ds(off, RB)])
            return carry

        lax.fori_loop(0, n_chunks, chunk, 0)

    return sc_kernel


def kernel(question_ids, concept_ids, question_table, concept_table,
           position_table):
    B, S = question_ids.shape
    N = B * S
    qf = question_ids.reshape(N).astype(jnp.int32)
    cf = concept_ids.reshape(N).astype(jnp.int32)
    pf = jnp.tile(jnp.arange(S, dtype=jnp.int32), B)
    out = _build_sc_kernel(N, 128)(qf, cf, pf, question_table,
                                   concept_table, position_table)
    return out.reshape(B, S, _H)


# speedup vs baseline: 3.6640x; 3.6640x over previous
import functools

import jax
import jax.numpy as jnp
from jax import lax
from jax.experimental import pallas as pl
from jax.experimental.pallas import tpu as pltpu
from jax.experimental.pallas import tpu_sc as plsc

_H = 64


@functools.lru_cache(maxsize=None)
def _build_sc_kernel(N, RB):
    NW = 32
    per_w = N // NW
    n_chunks = per_w // RB
    assert per_w % RB == 0

    mesh = plsc.VectorSubcoreMesh(core_axis_name="c", subcore_axis_name="s")

    @functools.partial(
        pl.kernel,
        mesh=mesh,
        out_type=jax.ShapeDtypeStruct((N, _H), jnp.float32),
        scratch_types=[
            pltpu.VMEM((RB,), jnp.int32),
            pltpu.VMEM((RB,), jnp.int32),
            pltpu.VMEM((RB,), jnp.int32),
            pltpu.VMEM((RB, _H), jnp.float32),
            pltpu.SemaphoreType.DMA,
        ],
        compiler_params=pltpu.CompilerParams(use_tc_tiling_on_sc=False),
    )
    def sc_kernel(qid, cid, pid, qtab, ctab, ptab, out, qi_v, ci_v, pi_v,
                  buf, sem):
        wid = lax.axis_index("s") * 2 + lax.axis_index("c")
        base = wid * per_w

        def chunk(g, carry):
            off = base + g * RB
            pltpu.sync_copy(qid.at[pl.ds(off, RB)], qi_v)
            pltpu.sync_copy(cid.at[pl.ds(off, RB)], ci_v)
            pltpu.sync_copy(pid.at[pl.ds(off, RB)], pi_v)
            pltpu.async_copy(ctab.at[ci_v], buf, sem).wait()
            pltpu.async_copy(qtab.at[qi_v], buf, sem, add=True).wait()
            pltpu.async_copy(ptab.at[pi_v], buf, sem, add=True).wait()
            pltpu.sync_copy(buf, out.at[pl.ds(off, RB)])
            return carry

        lax.fori_loop(0, n_chunks, chunk, 0)

    return sc_kernel


def kernel(question_ids, concept_ids, question_table, concept_table,
           position_table):
    B, S = question_ids.shape
    N = B * S
    qf = question_ids.reshape(N).astype(jnp.int32)
    cf = concept_ids.reshape(N).astype(jnp.int32)
    pf = jnp.tile(jnp.arange(S, dtype=jnp.int32), B)
    out = _build_sc_kernel(N, 128)(qf, cf, pf, question_table,
                                   concept_table, position_table)
    return out.reshape(B, S, _H)


# speedup vs baseline: 4.5922x; 1.2533x over previous
import functools

import jax
import jax.numpy as jnp
from jax import lax
from jax.experimental import pallas as pl
from jax.experimental.pallas import tpu as pltpu
from jax.experimental.pallas import tpu_sc as plsc

_H = 64


@functools.lru_cache(maxsize=None)
def _build_sc_kernel(N, RB):
    NW = 32
    per_w = N // NW
    n_chunks = per_w // RB
    assert per_w % RB == 0

    mesh = plsc.VectorSubcoreMesh(core_axis_name="c", subcore_axis_name="s")

    @functools.partial(
        pl.kernel,
        mesh=mesh,
        out_type=jax.ShapeDtypeStruct((N, _H), jnp.float32),
        scratch_types=[
            pltpu.VMEM((RB,), jnp.int32),
            pltpu.VMEM((RB,), jnp.int32),
            pltpu.VMEM((RB,), jnp.int32),
            pltpu.VMEM((RB, _H), jnp.float32),
            pltpu.SemaphoreType.DMA,
        ],
        compiler_params=pltpu.CompilerParams(use_tc_tiling_on_sc=False),
    )
    def sc_kernel(qid, cid, pid, qtab, ctab, ptab, out, qi_v, ci_v, pi_v,
                  buf, sem):
        wid = lax.axis_index("s") * 2 + lax.axis_index("c")
        base = wid * per_w

        def chunk(g, carry):
            off = base + g * RB
            pltpu.sync_copy(qid.at[pl.ds(off, RB)], qi_v)
            pltpu.sync_copy(cid.at[pl.ds(off, RB)], ci_v)
            pltpu.sync_copy(pid.at[pl.ds(off, RB)], pi_v)
            pltpu.async_copy(ctab.at[ci_v], buf, sem).wait()
            pltpu.async_copy(qtab.at[qi_v], buf, sem, add=True).wait()
            pltpu.async_copy(ptab.at[pi_v], buf, sem, add=True).wait()
            pltpu.sync_copy(buf, out.at[pl.ds(off, RB)])
            return carry

        lax.fori_loop(0, n_chunks, chunk, 0)

    return sc_kernel


def kernel(question_ids, concept_ids, question_table, concept_table,
           position_table):
    B, S = question_ids.shape
    N = B * S
    qf = question_ids.reshape(N).astype(jnp.int32)
    cf = concept_ids.reshape(N).astype(jnp.int32)
    pf = jnp.tile(jnp.arange(S, dtype=jnp.int32), B)
    out = _build_sc_kernel(N, 512)(qf, cf, pf, question_table,
                                   concept_table, position_table)
    return out.reshape(B, S, _H)
